# baseline (speedup 1.0000x reference)
import functools

import jax
import jax.numpy as jnp
from jax import lax
from jax.experimental import pallas as pl
from jax.experimental.pallas import tpu as pltpu
from jax.experimental.pallas import tpu_sc as plsc

_BATCH = 16384
_SUB_DIM = 32
_CHUNK = 128


def _build(batch, sub_dim):
    info = plsc.get_sparse_core_info()
    nw = info.num_cores * info.num_subcores
    bw = batch // nw
    nchunk = bw // _CHUNK
    idx_rows = batch // _CHUNK
    mesh = plsc.VectorSubcoreMesh(core_axis_name="c", subcore_axis_name="s")

    @functools.partial(
        pl.kernel,
        mesh=mesh,
        compiler_params=pltpu.CompilerParams(use_tc_tiling_on_sc=False),
        out_type=jax.ShapeDtypeStruct((batch, 2 * sub_dim), jnp.float32),
        scratch_types=[
            pltpu.VMEM((nchunk, _CHUNK), jnp.int32),
            pltpu.VMEM((nchunk, _CHUNK), jnp.int32),
            pltpu.VMEM((bw, sub_dim), jnp.float32),
            pltpu.VMEM((bw, sub_dim), jnp.float32),
            pltpu.SemaphoreType.DMA,
        ],
    )
    def hash_embed(idx_hbm, t0_hbm, t1_hbm, out_hbm, idx0_v, idx1_v, r0_v, r1_v, sem):
        wid = lax.axis_index("s") * info.num_cores + lax.axis_index("c")
        base = wid * bw
        pltpu.sync_copy(idx_hbm.at[pl.ds(wid * nchunk, nchunk)], idx0_v)
        pltpu.sync_copy(idx_hbm.at[pl.ds(idx_rows + wid * nchunk, nchunk)], idx1_v)
        copies = []
        for j in range(nchunk):
            copies.append(
                pltpu.async_copy(
                    t0_hbm.at[idx0_v.at[j]],
                    r0_v.at[pl.ds(j * _CHUNK, _CHUNK)],
                    sem,
                )
            )
            copies.append(
                pltpu.async_copy(
                    t1_hbm.at[idx1_v.at[j]],
                    r1_v.at[pl.ds(j * _CHUNK, _CHUNK)],
                    sem,
                )
            )
        for c in copies:
            c.wait()
        pltpu.sync_copy(r0_v, out_hbm.at[pl.ds(base, bw), pl.ds(0, sub_dim)])
        pltpu.sync_copy(r1_v, out_hbm.at[pl.ds(base, bw), pl.ds(sub_dim, sub_dim)])

    return hash_embed


_hash_embed = _build(_BATCH, _SUB_DIM)


def kernel(indices, table0, table1):
    idx = indices.astype(jnp.int32).reshape(2 * _BATCH // _CHUNK, _CHUNK)
    return _hash_embed(idx, table0, table1)
